# baseline (speedup 1.0000x reference)
import functools

import jax
import jax.numpy as jnp
from jax import lax
from jax.experimental import pallas as pl
from jax.experimental.pallas import tpu as pltpu
from jax.experimental.pallas import tpu_sc as plsc

_SLEN = 20
_VOCAB = 100000
_PAD = 32
_LANES = 16


def _log16(x):
    bits = lax.bitcast_convert_type(x, jnp.int32)
    e = (bits >> 23) - 127
    m = lax.bitcast_convert_type((bits & 0x007FFFFF) | 0x3F800000, jnp.float32)
    big = m > 1.41421356237
    m = jnp.where(big, m * 0.5, m)
    ef = jnp.where(big, e + 1, e).astype(jnp.float32)
    t = m - 1.0
    z = t * t
    p = jnp.full((_LANES,), 7.0376836292e-2, jnp.float32)
    for c in (-1.1514610310e-1, 1.1676998740e-1, -1.2420140846e-1,
              1.4249322787e-1, -1.6668057665e-1, 2.0000714765e-1,
              -2.4999993993e-1, 3.3333331174e-1):
        p = p * t + c
    y = t * z * p - 0.5 * z
    return t + y + ef * 0.69314718056


@functools.partial(
    pl.kernel,
    out_type=jax.ShapeDtypeStruct((_LANES,), jnp.float32),
    mesh=plsc.VectorSubcoreMesh(core_axis_name="c", subcore_axis_name="s"),
    compiler_params=pltpu.CompilerParams(use_tc_tiling_on_sc=True),
    scratch_types=[
        pltpu.VMEM((_PAD,), jnp.int32),
        pltpu.VMEM((_SLEN, 8, 128), jnp.float32),
        pltpu.VMEM((_LANES,), jnp.float32),
        pltpu.SemaphoreType.DMA,
    ],
)
def _pll_score(probs_hbm, ids_hbm, out_hbm, ids_v, tiles_v, out_v, sem):
    cid = lax.axis_index("c")
    sid = lax.axis_index("s")

    @pl.when(jnp.logical_and(cid == 0, sid == 0))
    def _():
        pltpu.sync_copy(ids_hbm, ids_v)
        iv = [ids_v[pl.ds(0, _LANES)], ids_v[pl.ds(_LANES, _LANES)]]
        copies = []
        for k in range(_SLEN):
            idv = iv[k // _LANES][k % _LANES]
            c0 = pl.multiple_of(
                (jnp.maximum(idv, 0) >> 7) << 7, 128)
            r0 = ((k + 1) // 8) * 8
            nr = min(8, _SLEN + 1 - r0)
            copies.append(pltpu.async_copy(
                probs_hbm.at[k, pl.ds(r0, nr), pl.ds(c0, 128)],
                tiles_v.at[k, pl.ds(0, nr)], sem))
        for c in copies:
            c.wait()

        kio = lax.iota(jnp.int32, _LANES)
        lgsum = jnp.zeros((_LANES,), jnp.float32)
        count = jnp.float32(0.0)
        for g in range(4):
            acc = jnp.ones((_LANES,), jnp.float32)
            for k in range(g * 5, g * 5 + 5):
                idv = iv[k // _LANES][k % _LANES]
                valid = idv >= 0
                off = jnp.where(valid, jnp.maximum(idv, 0) & 127, 128)
                row = (k + 1) % 8
                for c in range(8):
                    x = tiles_v[k, row, pl.ds(c * 16, _LANES)]
                    sel = (c * 16 + kio) == off
                    acc = acc * jnp.where(sel, x, 1.0)
                count = count + jnp.where(valid, 1.0, 0.0)
            lgsum = lgsum + _log16(acc)
        total = lgsum[0]
        for i in range(1, _LANES):
            total = total + lgsum[i]
        out_v[...] = (jnp.full((_LANES,), total, jnp.float32) /
                      jnp.full((_LANES,), count, jnp.float32))
        pltpu.sync_copy(out_v, out_hbm)


def kernel(probs, origids):
    ids = jnp.full((_PAD,), -1, jnp.int32).at[:_SLEN].set(
        origids.astype(jnp.int32))
    probs_t = jnp.transpose(probs, (1, 0, 2))
    out = _pll_score(probs_t, ids)
    return out[0]


# speedup vs baseline: 9.2283x; 9.2283x over previous
import jax
import jax.numpy as jnp
from jax import lax
from jax.experimental import pallas as pl
from jax.experimental.pallas import tpu as pltpu

_SLEN = 20
_B = 128
_ROWS = 8
_GRP = 5


def _body(ids_ref, probs_ref, out_ref, tiles, sem):
    copies = []
    for k in range(_SLEN):
        idv = ids_ref[k]
        c0 = pl.multiple_of((jnp.maximum(idv, 0) >> 7) << 7, _B)
        r0 = ((k + 1) // _ROWS) * _ROWS
        nr = min(_ROWS, _SLEN + 1 - r0)
        copies.append(pltpu.make_async_copy(
            probs_ref.at[k, pl.ds(r0, nr), pl.ds(c0, _B)],
            tiles.at[k, pl.ds(0, nr)], sem))
    for c in copies:
        c.start()
    for c in copies:
        c.wait()

    rows = lax.broadcasted_iota(jnp.int32, (_ROWS, _B), 0)
    lanes = lax.broadcasted_iota(jnp.int32, (_ROWS, _B), 1)
    acc = jnp.zeros((_ROWS, _B), jnp.float32)
    cnt = jnp.float32(0.0)
    for g in range(_SLEN // _GRP):
        v = jnp.ones((_ROWS, _B), jnp.float32)
        for k in range(g * _GRP, (g + 1) * _GRP):
            idv = ids_ref[k]
            valid = idv >= 0
            off = lax.rem(jnp.maximum(idv, 0), _B)
            sel = jnp.logical_and(
                jnp.logical_and(rows == (k + 1) % _ROWS, lanes == off), valid)
            v = v * jnp.where(sel, tiles[k], 1.0)
            cnt = cnt + jnp.where(valid, 1.0, 0.0)
        acc = acc + jnp.log(v)
    out_ref[0] = jnp.sum(acc) / cnt


def kernel(probs, origids):
    ids = origids.astype(jnp.int32)
    probs_t = jnp.transpose(probs, (1, 0, 2))
    grid_spec = pltpu.PrefetchScalarGridSpec(
        num_scalar_prefetch=1,
        grid=(1,),
        in_specs=[pl.BlockSpec(memory_space=pl.ANY)],
        out_specs=pl.BlockSpec(memory_space=pltpu.SMEM),
        scratch_shapes=[
            pltpu.VMEM((_SLEN, _ROWS, _B), jnp.float32),
            pltpu.SemaphoreType.DMA,
        ],
    )
    out = pl.pallas_call(
        _body,
        grid_spec=grid_spec,
        out_shape=jax.ShapeDtypeStruct((1,), jnp.float32),
    )(ids, probs_t)
    return out[0]


# speedup vs baseline: 9.4807x; 1.0274x over previous
import jax
import jax.numpy as jnp
from jax import lax
from jax.experimental import pallas as pl
from jax.experimental.pallas import tpu as pltpu

_SLEN = 20
_B = 128
_GRP = 5


def _body(ids_ref, probs_ref, out_ref, rows_v, sem):
    copies = []
    for k in range(_SLEN):
        idv = ids_ref[k]
        c0 = pl.multiple_of((jnp.maximum(idv, 0) >> 7) << 7, _B)
        copies.append(pltpu.make_async_copy(
            probs_ref.at[k, pl.ds(k + 1, 1), pl.ds(c0, _B)],
            rows_v.at[k], sem))
    for c in copies:
        c.start()
    for c in copies:
        c.wait()

    lanes = lax.broadcasted_iota(jnp.int32, (1, _B), 1)
    acc = jnp.zeros((1, _B), jnp.float32)
    cnt = jnp.float32(0.0)
    for g in range(_SLEN // _GRP):
        v = jnp.ones((1, _B), jnp.float32)
        for k in range(g * _GRP, (g + 1) * _GRP):
            idv = ids_ref[k]
            valid = idv >= 0
            off = lax.rem(jnp.maximum(idv, 0), _B)
            sel = jnp.logical_and(lanes == off, valid)
            v = v * jnp.where(sel, rows_v[k], 1.0)
            cnt = cnt + jnp.where(valid, 1.0, 0.0)
        acc = acc + jnp.log(v)
    out_ref[0] = jnp.sum(acc) / cnt


def kernel(probs, origids):
    ids = origids.astype(jnp.int32)
    probs_t = jnp.transpose(probs, (1, 0, 2))
    grid_spec = pltpu.PrefetchScalarGridSpec(
        num_scalar_prefetch=1,
        grid=(1,),
        in_specs=[pl.BlockSpec(memory_space=pl.ANY)],
        out_specs=pl.BlockSpec(memory_space=pltpu.SMEM),
        scratch_shapes=[
            pltpu.VMEM((_SLEN, 1, _B), jnp.float32),
            pltpu.SemaphoreType.DMA,
        ],
    )
    out = pl.pallas_call(
        _body,
        grid_spec=grid_spec,
        out_shape=jax.ShapeDtypeStruct((1,), jnp.float32),
    )(ids, probs_t)
    return out[0]
